# baseline (speedup 1.0000x reference)
import functools

import jax
import jax.numpy as jnp
from jax import lax
from jax.experimental import pallas as pl
from jax.experimental.pallas import tpu as pltpu
from jax.experimental.pallas import tpu_sc as plsc

N = 10000
E = 320000
G = 16
D = 128

NC = 2
NS = 16
NW = NC * NS
B = 64
EPW = 10240
PAD = EPW - E // NW
K = EPW // B
NB = 5
SLAB = 20
NSLAB = K // SLAB
NA = N + 8
DB = 125
DK = E // (NW * DB)
ROWS_PER_TILE = N // NS
DEG_CHUNK = 1000


def _sc_mesh():
    return plsc.VectorSubcoreMesh(core_axis_name="c", subcore_axis_name="s")


def _deg_body(dst_hbm, ones_hbm, zeros_hbm, out_hbm, acc, dst_v, ones_v,
              zbuf, sem):
    cid = lax.axis_index("c")
    sid = lax.axis_index("s")
    wid = sid * NC + cid
    @pl.when(sid < 10)
    def _():
        pltpu.sync_copy(zeros_hbm.at[pl.ds(sid * DEG_CHUNK, DEG_CHUNK)], zbuf)
        pltpu.sync_copy(zbuf, acc.at[pl.ds(sid * DEG_CHUNK, DEG_CHUNK)])
    pltpu.sync_copy(dst_hbm.at[wid], dst_v)
    pltpu.sync_copy(ones_hbm, ones_v)
    plsc.subcore_barrier()

    def chunk(j, carry):
        pltpu.async_copy(ones_v, acc.at[dst_v.at[j]], sem, add=True).wait()
        return carry

    lax.fori_loop(0, DK, chunk, 0)
    plsc.subcore_barrier()
    @pl.when(sid < 10)
    def _():
        pltpu.sync_copy(acc.at[pl.ds(sid * DEG_CHUNK, DEG_CHUNK)], zbuf)
        pltpu.sync_copy(zbuf,
                        out_hbm.at[pl.ds(cid * N + sid * DEG_CHUNK, DEG_CHUNK)])


def _sc_degree(dst3, ones_b, zeros_n):
    f = pl.kernel(
        _deg_body,
        out_type=jax.ShapeDtypeStruct((NC * N,), jnp.float32),
        mesh=_sc_mesh(),
        scratch_types=[
            pltpu.VMEM_SHARED((N,), jnp.float32),
            pltpu.VMEM((DK, DB), jnp.int32),
            pltpu.VMEM((DB,), jnp.float32),
            pltpu.VMEM((DEG_CHUNK,), jnp.float32),
            pltpu.SemaphoreType.DMA,
        ],
    )
    return f(dst3, ones_b, zeros_n)


def _agg_body(src_hbm, dst_hbm, y_hbm, zeros_hbm, out_hbm,
              acc, src_v, dst_v, b0, b1, b2, b3, b4, m0, m1, m2, m3, m4):
    bufs = (b0, b1, b2, b3, b4)
    sems = (m0, m1, m2, m3, m4)
    cid = lax.axis_index("c")
    sid = lax.axis_index("s")
    wid = sid * NC + cid
    r0 = sid * DEG_CHUNK
    @pl.when(sid < 10)
    def _():
        pltpu.sync_copy(zeros_hbm.at[pl.ds(0, 40)], b0.at[pl.ds(0, 40)])
        for i in range(25):
            pltpu.sync_copy(b0.at[pl.ds(0, 40)],
                            acc.at[pl.ds(r0 + i * 40, 40)])
    plsc.subcore_barrier()

    def g_start(j, i):
        pltpu.async_copy(y_hbm.at[src_v.at[j]], bufs[i], sems[i])

    def g_wait(j, i):
        pltpu.make_async_copy(y_hbm.at[src_v.at[j]], bufs[i], sems[i]).wait()

    def s_start(j, i):
        pltpu.async_copy(bufs[i], acc.at[dst_v.at[j]], sems[i], add=True)

    def s_wait(j, i):
        pltpu.make_async_copy(bufs[i], acc.at[dst_v.at[j]], sems[i]).wait()

    for h in range(NSLAB):
        pltpu.sync_copy(src_hbm.at[wid, h], src_v)
        pltpu.sync_copy(dst_hbm.at[wid, h], dst_v)
        g_start(0, 0)
        g_start(1, 1)

        def rot(m, carry):
            for i in range(NB):
                j = NB * m + i
                iw = (i + 2) % NB
                if i < NB - 2:
                    @pl.when(m > 0)
                    def _(j=j, iw=iw):
                        s_wait(j - (NB - 2), iw)
                    g_start(j + 2, iw)
                else:
                    s_wait(j - (NB - 2), iw)
                    @pl.when(m < SLAB // NB - 1)
                    def _(j=j, iw=iw):
                        g_start(j + 2, iw)
                g_wait(j, i)
                s_start(j, i)
            return carry

        lax.fori_loop(0, SLAB // NB, rot, 0)
        for t in range(NB - 2):
            j = SLAB - (NB - 2) + t
            s_wait(j, j % NB)

    plsc.subcore_barrier()
    @pl.when(sid < 10)
    def _():
        for i in range(25):
            pltpu.sync_copy(acc.at[pl.ds(r0 + i * 40, 40)],
                            b0.at[pl.ds(0, 40)])
            pltpu.sync_copy(b0.at[pl.ds(0, 40)],
                            out_hbm.at[cid, pl.ds(r0 + i * 40, 40)])


def _sc_agg(src3, dst3, y, zeros_nd):
    f = pl.kernel(
        _agg_body,
        out_type=jax.ShapeDtypeStruct((NC, N, D), jnp.float32),
        mesh=_sc_mesh(),
        scratch_types=[
            pltpu.VMEM_SHARED((NA, D), jnp.float32),
            pltpu.VMEM((SLAB, B), jnp.int32),
            pltpu.VMEM((SLAB, B), jnp.int32),
            pltpu.VMEM((B, D), jnp.float32),
            pltpu.VMEM((B, D), jnp.float32),
            pltpu.VMEM((B, D), jnp.float32),
            pltpu.VMEM((B, D), jnp.float32),
            pltpu.VMEM((B, D), jnp.float32),
            pltpu.SemaphoreType.DMA,
            pltpu.SemaphoreType.DMA,
            pltpu.SemaphoreType.DMA,
            pltpu.SemaphoreType.DMA,
            pltpu.SemaphoreType.DMA,
        ],
    )
    return f(src3, dst3, y, zeros_nd)


_RB = 1000


def _first_body(degT_ref, x_ref, W_ref, dis_ref, y_ref):
    deg = degT_ref[:, 0:1] + degT_ref[:, 1:2] + 1.0
    dis = lax.rsqrt(deg)
    dis_ref[...] = dis
    y_ref[...] = dis * jnp.dot(x_ref[...], W_ref[...],
                               preferred_element_type=jnp.float32)


def _tc_first(degT, x, W1):
    grid = N // _RB
    return pl.pallas_call(
        _first_body,
        grid=(grid,),
        in_specs=[
            pl.BlockSpec((_RB, NC), lambda i: (i, 0)),
            pl.BlockSpec((_RB, D), lambda i: (i, 0)),
            pl.BlockSpec((D, D), lambda i: (0, 0)),
        ],
        out_specs=[
            pl.BlockSpec((_RB, 1), lambda i: (i, 0)),
            pl.BlockSpec((_RB, D), lambda i: (i, 0)),
        ],
        out_shape=[
            jax.ShapeDtypeStruct((N, 1), jnp.float32),
            jax.ShapeDtypeStruct((N, D), jnp.float32),
        ],
    )(degT, x, W1)


def _mid_body(agg_ref, y_ref, dis_ref, b_ref, W_ref, out_ref):
    s = agg_ref[0] + agg_ref[1] + y_ref[...]
    h = jnp.maximum(dis_ref[...] * s + b_ref[...], 0.0)
    out_ref[...] = dis_ref[...] * jnp.dot(h, W_ref[...],
                                          preferred_element_type=jnp.float32)


def _tc_mid(aggP, y, dis, b_row, W):
    grid = N // _RB
    return pl.pallas_call(
        _mid_body,
        grid=(grid,),
        in_specs=[
            pl.BlockSpec((NC, _RB, D), lambda i: (0, i, 0)),
            pl.BlockSpec((_RB, D), lambda i: (i, 0)),
            pl.BlockSpec((_RB, 1), lambda i: (i, 0)),
            pl.BlockSpec((1, D), lambda i: (0, 0)),
            pl.BlockSpec((D, D), lambda i: (0, 0)),
        ],
        out_specs=pl.BlockSpec((_RB, D), lambda i: (i, 0)),
        out_shape=jax.ShapeDtypeStruct((N, D), jnp.float32),
    )(aggP, y, dis, b_row, W)


def _final_body(agg_ref, y_ref, dis_ref, b_ref, batch_ref, hlr_ref, std_ref,
                Wf1a_ref, Wf1b_ref, Wf1c_ref, bf1_ref, Wf2_ref, bf2_ref,
                out_ref):
    s = agg_ref[0] + agg_ref[1] + y_ref[...]
    h = jnp.maximum(dis_ref[...] * s + b_ref[...], 0.0)
    gids = lax.broadcasted_iota(jnp.int32, (G, N), 0)
    mask = (gids == batch_ref[...]).astype(jnp.float32)
    sums = jnp.dot(mask, h, preferred_element_type=jnp.float32)
    cnt = jnp.dot(mask, jnp.ones((N, 1), jnp.float32),
                  preferred_element_type=jnp.float32)
    pooled = sums / jnp.maximum(cnt, 1.0)
    z = (jnp.dot(pooled, Wf1a_ref[...], preferred_element_type=jnp.float32)
         + hlr_ref[...] * Wf1b_ref[...]
         + std_ref[...] * Wf1c_ref[...]
         + bf1_ref[...])
    z = jnp.maximum(z, 0.0)
    out_ref[...] = (jnp.dot(z, Wf2_ref[...], preferred_element_type=jnp.float32)
                    + bf2_ref[...])


def _tc_final(aggP, y, dis, b_row, batch_row, hlr, std,
              Wf1a, Wf1b, Wf1c, bf1_row, Wf2, bf2_row):
    return pl.pallas_call(
        _final_body,
        out_shape=jax.ShapeDtypeStruct((G, D), jnp.float32),
    )(aggP, y, dis, b_row, batch_row, hlr, std,
      Wf1a, Wf1b, Wf1c, bf1_row, Wf2, bf2_row)


def kernel(x, edge_index, batch, hlr, std,
           W1, b1, W2, b2, W3, b3, Wf1, bf1, Wf2, bf2):
    src2 = edge_index[0].reshape(NW, E // NW)
    dst2 = edge_index[1].reshape(NW, E // NW)
    lane = jnp.arange(PAD, dtype=jnp.int32)[None, :]
    wrow = jnp.arange(NW, dtype=jnp.int32)[:, None]
    pad_src = ((wrow * PAD + lane) * 37) % N
    pad_dst = N + (lane + wrow) % 8
    src3 = jnp.concatenate([src2, pad_src], axis=1).reshape(NW, NSLAB, SLAB, B)
    dst3 = jnp.concatenate([dst2, pad_dst + jnp.zeros_like(pad_src)],
                           axis=1).reshape(NW, NSLAB, SLAB, B)
    dst3d = edge_index[1].reshape(NW, DK, DB)
    zeros_nd = jnp.zeros((N, D), jnp.float32)
    zeros_n = jnp.zeros((N,), jnp.float32)
    ones_b = jnp.ones((DB,), jnp.float32)

    degP = _sc_degree(dst3d, ones_b, zeros_n).reshape(NC, N)
    dis, y1 = _tc_first(degP.T, x, W1)

    a1 = _sc_agg(src3, dst3, y1, zeros_nd)
    y2 = _tc_mid(a1, y1, dis, b1.reshape(1, D), W2)
    a2 = _sc_agg(src3, dst3, y2, zeros_nd)
    y3 = _tc_mid(a2, y2, dis, b2.reshape(1, D), W3)
    a3 = _sc_agg(src3, dst3, y3, zeros_nd)

    out = _tc_final(a3, y3, dis, b3.reshape(1, D), batch.reshape(1, N),
                    hlr, std,
                    Wf1[:D], Wf1[D:D + 1], Wf1[D + 1:D + 2],
                    bf1.reshape(1, D), Wf2, bf2.reshape(1, D))
    return out


# speedup vs baseline: 1.0531x; 1.0531x over previous
import functools

import jax
import jax.numpy as jnp
from jax import lax
from jax.experimental import pallas as pl
from jax.experimental.pallas import tpu as pltpu
from jax.experimental.pallas import tpu_sc as plsc

N = 10000
E = 320000
G = 16
D = 128

NC = 2
NS = 16
NW = NC * NS
B = 80
EPW = 10240
PAD = EPW - E // NW
K = EPW // B
NB = 4
SLAB = 32
NSLAB = K // SLAB
NA = N + 8
DB = 125
DK = E // (NW * DB)
ROWS_PER_TILE = N // NS
DEG_CHUNK = 1000


def _sc_mesh():
    return plsc.VectorSubcoreMesh(core_axis_name="c", subcore_axis_name="s")


def _deg_body(dst_hbm, ones_hbm, zeros_hbm, out_hbm, acc, dst_v, ones_v,
              zbuf, sem):
    cid = lax.axis_index("c")
    sid = lax.axis_index("s")
    wid = sid * NC + cid
    @pl.when(sid < 10)
    def _():
        pltpu.sync_copy(zeros_hbm.at[pl.ds(sid * DEG_CHUNK, DEG_CHUNK)], zbuf)
        pltpu.sync_copy(zbuf, acc.at[pl.ds(sid * DEG_CHUNK, DEG_CHUNK)])
    pltpu.sync_copy(dst_hbm.at[wid], dst_v)
    pltpu.sync_copy(ones_hbm, ones_v)
    plsc.subcore_barrier()

    def chunk(j, carry):
        pltpu.async_copy(ones_v, acc.at[dst_v.at[j]], sem, add=True).wait()
        return carry

    lax.fori_loop(0, DK, chunk, 0)
    plsc.subcore_barrier()
    @pl.when(sid < 10)
    def _():
        pltpu.sync_copy(acc.at[pl.ds(sid * DEG_CHUNK, DEG_CHUNK)], zbuf)
        pltpu.sync_copy(zbuf,
                        out_hbm.at[pl.ds(cid * N + sid * DEG_CHUNK, DEG_CHUNK)])


def _sc_degree(dst3, ones_b, zeros_n):
    f = pl.kernel(
        _deg_body,
        out_type=jax.ShapeDtypeStruct((NC * N,), jnp.float32),
        mesh=_sc_mesh(),
        scratch_types=[
            pltpu.VMEM_SHARED((N,), jnp.float32),
            pltpu.VMEM((DK, DB), jnp.int32),
            pltpu.VMEM((DB,), jnp.float32),
            pltpu.VMEM((DEG_CHUNK,), jnp.float32),
            pltpu.SemaphoreType.DMA,
        ],
    )
    return f(dst3, ones_b, zeros_n)


def _agg_body(src_hbm, dst_hbm, y_hbm, zeros_hbm, out_hbm,
              acc, src_v, dst_v, b0, b1, b2, b3, m0, m1, m2, m3):
    bufs = (b0, b1, b2, b3)
    sems = (m0, m1, m2, m3)
    cid = lax.axis_index("c")
    sid = lax.axis_index("s")
    wid = sid * NC + cid
    r0 = sid * DEG_CHUNK
    @pl.when(sid < 10)
    def _():
        pltpu.sync_copy(zeros_hbm.at[pl.ds(0, 40)], b0.at[pl.ds(0, 40)])
        for i in range(25):
            pltpu.sync_copy(b0.at[pl.ds(0, 40)],
                            acc.at[pl.ds(r0 + i * 40, 40)])
    plsc.subcore_barrier()

    def g_start(j, i):
        pltpu.async_copy(y_hbm.at[src_v.at[j]], bufs[i], sems[i])

    def g_wait(j, i):
        pltpu.make_async_copy(y_hbm.at[src_v.at[j]], bufs[i], sems[i]).wait()

    def s_start(j, i):
        pltpu.async_copy(bufs[i], acc.at[dst_v.at[j]], sems[i], add=True)

    def s_wait(j, i):
        pltpu.make_async_copy(bufs[i], acc.at[dst_v.at[j]], sems[i]).wait()

    for h in range(NSLAB):
        pltpu.sync_copy(src_hbm.at[wid, h], src_v)
        pltpu.sync_copy(dst_hbm.at[wid, h], dst_v)
        g_start(0, 0)
        g_start(1, 1)

        def rot(m, carry):
            for i in range(NB):
                j = NB * m + i
                iw = (i + 2) % NB
                if i < NB - 2:
                    @pl.when(m > 0)
                    def _(j=j, iw=iw):
                        s_wait(j - (NB - 2), iw)
                    g_start(j + 2, iw)
                else:
                    s_wait(j - (NB - 2), iw)
                    @pl.when(m < SLAB // NB - 1)
                    def _(j=j, iw=iw):
                        g_start(j + 2, iw)
                g_wait(j, i)
                s_start(j, i)
            return carry

        lax.fori_loop(0, SLAB // NB, rot, 0)
        for t in range(NB - 2):
            j = SLAB - (NB - 2) + t
            s_wait(j, j % NB)

    plsc.subcore_barrier()
    @pl.when(sid < 10)
    def _():
        for i in range(25):
            pltpu.sync_copy(acc.at[pl.ds(r0 + i * 40, 40)],
                            b0.at[pl.ds(0, 40)])
            pltpu.sync_copy(b0.at[pl.ds(0, 40)],
                            out_hbm.at[cid, pl.ds(r0 + i * 40, 40)])


def _sc_agg(src3, dst3, y, zeros_nd):
    f = pl.kernel(
        _agg_body,
        out_type=jax.ShapeDtypeStruct((NC, N, D), jnp.float32),
        mesh=_sc_mesh(),
        scratch_types=[
            pltpu.VMEM_SHARED((NA, D), jnp.float32),
            pltpu.VMEM((SLAB, B), jnp.int32),
            pltpu.VMEM((SLAB, B), jnp.int32),
            pltpu.VMEM((B, D), jnp.float32),
            pltpu.VMEM((B, D), jnp.float32),
            pltpu.VMEM((B, D), jnp.float32),
            pltpu.VMEM((B, D), jnp.float32),
            pltpu.SemaphoreType.DMA,
            pltpu.SemaphoreType.DMA,
            pltpu.SemaphoreType.DMA,
            pltpu.SemaphoreType.DMA,
        ],
    )
    return f(src3, dst3, y, zeros_nd)


_RB = 1000


def _first_body(degT_ref, x_ref, W_ref, dis_ref, y_ref):
    deg = degT_ref[:, 0:1] + degT_ref[:, 1:2] + 1.0
    dis = lax.rsqrt(deg)
    dis_ref[...] = dis
    y_ref[...] = dis * jnp.dot(x_ref[...], W_ref[...],
                               preferred_element_type=jnp.float32)


def _tc_first(degT, x, W1):
    grid = N // _RB
    return pl.pallas_call(
        _first_body,
        grid=(grid,),
        in_specs=[
            pl.BlockSpec((_RB, NC), lambda i: (i, 0)),
            pl.BlockSpec((_RB, D), lambda i: (i, 0)),
            pl.BlockSpec((D, D), lambda i: (0, 0)),
        ],
        out_specs=[
            pl.BlockSpec((_RB, 1), lambda i: (i, 0)),
            pl.BlockSpec((_RB, D), lambda i: (i, 0)),
        ],
        out_shape=[
            jax.ShapeDtypeStruct((N, 1), jnp.float32),
            jax.ShapeDtypeStruct((N, D), jnp.float32),
        ],
    )(degT, x, W1)


def _mid_body(agg_ref, y_ref, dis_ref, b_ref, W_ref, out_ref):
    s = agg_ref[0] + agg_ref[1] + y_ref[...]
    h = jnp.maximum(dis_ref[...] * s + b_ref[...], 0.0)
    out_ref[...] = dis_ref[...] * jnp.dot(h, W_ref[...],
                                          preferred_element_type=jnp.float32)


def _tc_mid(aggP, y, dis, b_row, W):
    grid = N // _RB
    return pl.pallas_call(
        _mid_body,
        grid=(grid,),
        in_specs=[
            pl.BlockSpec((NC, _RB, D), lambda i: (0, i, 0)),
            pl.BlockSpec((_RB, D), lambda i: (i, 0)),
            pl.BlockSpec((_RB, 1), lambda i: (i, 0)),
            pl.BlockSpec((1, D), lambda i: (0, 0)),
            pl.BlockSpec((D, D), lambda i: (0, 0)),
        ],
        out_specs=pl.BlockSpec((_RB, D), lambda i: (i, 0)),
        out_shape=jax.ShapeDtypeStruct((N, D), jnp.float32),
    )(aggP, y, dis, b_row, W)


def _final_body(agg_ref, y_ref, dis_ref, b_ref, batch_ref, hlr_ref, std_ref,
                Wf1a_ref, Wf1b_ref, Wf1c_ref, bf1_ref, Wf2_ref, bf2_ref,
                out_ref):
    s = agg_ref[0] + agg_ref[1] + y_ref[...]
    h = jnp.maximum(dis_ref[...] * s + b_ref[...], 0.0)
    gids = lax.broadcasted_iota(jnp.int32, (G, N), 0)
    mask = (gids == batch_ref[...]).astype(jnp.float32)
    sums = jnp.dot(mask, h, preferred_element_type=jnp.float32)
    cnt = jnp.dot(mask, jnp.ones((N, 1), jnp.float32),
                  preferred_element_type=jnp.float32)
    pooled = sums / jnp.maximum(cnt, 1.0)
    z = (jnp.dot(pooled, Wf1a_ref[...], preferred_element_type=jnp.float32)
         + hlr_ref[...] * Wf1b_ref[...]
         + std_ref[...] * Wf1c_ref[...]
         + bf1_ref[...])
    z = jnp.maximum(z, 0.0)
    out_ref[...] = (jnp.dot(z, Wf2_ref[...], preferred_element_type=jnp.float32)
                    + bf2_ref[...])


def _tc_final(aggP, y, dis, b_row, batch_row, hlr, std,
              Wf1a, Wf1b, Wf1c, bf1_row, Wf2, bf2_row):
    return pl.pallas_call(
        _final_body,
        out_shape=jax.ShapeDtypeStruct((G, D), jnp.float32),
    )(aggP, y, dis, b_row, batch_row, hlr, std,
      Wf1a, Wf1b, Wf1c, bf1_row, Wf2, bf2_row)


def kernel(x, edge_index, batch, hlr, std,
           W1, b1, W2, b2, W3, b3, Wf1, bf1, Wf2, bf2):
    src2 = edge_index[0].reshape(NW, E // NW)
    dst2 = edge_index[1].reshape(NW, E // NW)
    lane = jnp.arange(PAD, dtype=jnp.int32)[None, :]
    wrow = jnp.arange(NW, dtype=jnp.int32)[:, None]
    pad_src = ((wrow * PAD + lane) * 37) % N
    pad_dst = N + (lane + wrow) % 8
    src3 = jnp.concatenate([src2, pad_src], axis=1).reshape(NW, NSLAB, SLAB, B)
    dst3 = jnp.concatenate([dst2, pad_dst + jnp.zeros_like(pad_src)],
                           axis=1).reshape(NW, NSLAB, SLAB, B)
    dst3d = edge_index[1].reshape(NW, DK, DB)
    zeros_nd = jnp.zeros((N, D), jnp.float32)
    zeros_n = jnp.zeros((N,), jnp.float32)
    ones_b = jnp.ones((DB,), jnp.float32)

    degP = _sc_degree(dst3d, ones_b, zeros_n).reshape(NC, N)
    dis, y1 = _tc_first(degP.T, x, W1)

    a1 = _sc_agg(src3, dst3, y1, zeros_nd)
    y2 = _tc_mid(a1, y1, dis, b1.reshape(1, D), W2)
    a2 = _sc_agg(src3, dst3, y2, zeros_nd)
    y3 = _tc_mid(a2, y2, dis, b2.reshape(1, D), W3)
    a3 = _sc_agg(src3, dst3, y3, zeros_nd)

    out = _tc_final(a3, y3, dis, b3.reshape(1, D), batch.reshape(1, N),
                    hlr, std,
                    Wf1[:D], Wf1[D:D + 1], Wf1[D + 1:D + 2],
                    bf1.reshape(1, D), Wf2, bf2.reshape(1, D))
    return out


# speedup vs baseline: 1.0605x; 1.0070x over previous
import functools

import jax
import jax.numpy as jnp
from jax import lax
from jax.experimental import pallas as pl
from jax.experimental.pallas import tpu as pltpu
from jax.experimental.pallas import tpu_sc as plsc

N = 10000
E = 320000
G = 16
D = 128

NC = 2
NS = 16
NW = NC * NS
B = 80
K = E // (NW * B)
NB = 4
SLAB = 25
NSLAB = K // SLAB
NQ = SLAB // NB
DB = 125
DK = E // (NW * DB)
ROWS_PER_TILE = N // NS
DEG_CHUNK = 1000


def _sc_mesh():
    return plsc.VectorSubcoreMesh(core_axis_name="c", subcore_axis_name="s")


def _deg_body(dst_hbm, ones_hbm, zeros_hbm, out_hbm, acc, dst_v, ones_v,
              zbuf, sem):
    cid = lax.axis_index("c")
    sid = lax.axis_index("s")
    wid = sid * NC + cid
    @pl.when(sid < 10)
    def _():
        pltpu.sync_copy(zeros_hbm, zbuf)
        pltpu.sync_copy(zbuf, acc.at[pl.ds(sid * DEG_CHUNK, DEG_CHUNK)])
    pltpu.sync_copy(dst_hbm.at[1, wid], dst_v)
    pltpu.sync_copy(ones_hbm, ones_v)
    plsc.subcore_barrier()

    def chunk(j, carry):
        pltpu.async_copy(ones_v, acc.at[dst_v.at[j]], sem, add=True).wait()
        return carry

    lax.fori_loop(0, DK, chunk, 0)
    plsc.subcore_barrier()
    @pl.when(sid < 10)
    def _():
        pltpu.sync_copy(acc.at[pl.ds(sid * DEG_CHUNK, DEG_CHUNK)], zbuf)
        pltpu.sync_copy(zbuf,
                        out_hbm.at[pl.ds(cid * N + sid * DEG_CHUNK, DEG_CHUNK)])


def _sc_degree(dst3, ones_b, zeros_n):
    f = pl.kernel(
        _deg_body,
        out_type=jax.ShapeDtypeStruct((NC * N,), jnp.float32),
        mesh=_sc_mesh(),
        scratch_types=[
            pltpu.VMEM_SHARED((N,), jnp.float32),
            pltpu.VMEM((DK, DB), jnp.int32),
            pltpu.VMEM((DB,), jnp.float32),
            pltpu.VMEM((DEG_CHUNK,), jnp.float32),
            pltpu.SemaphoreType.DMA,
        ],
    )
    return f(dst3, ones_b, zeros_n)


def _agg_body(ei_hbm, y_hbm, zeros_hbm, out_hbm,
              acc, src_v, dst_v, b0, b1, b2, b3, m0, m1, m2, m3):
    bufs = (b0, b1, b2, b3)
    sems = (m0, m1, m2, m3)
    cid = lax.axis_index("c")
    sid = lax.axis_index("s")
    wid = sid * NC + cid
    r0 = sid * DEG_CHUNK
    @pl.when(sid < 10)
    def _():
        pltpu.sync_copy(zeros_hbm, b0.at[pl.ds(0, 40)])
        for i in range(25):
            pltpu.sync_copy(b0.at[pl.ds(0, 40)],
                            acc.at[pl.ds(r0 + i * 40, 40)])
    plsc.subcore_barrier()

    def g_start(j, i):
        pltpu.async_copy(y_hbm.at[src_v.at[j]], bufs[i], sems[i])

    def g_wait(j, i):
        pltpu.make_async_copy(y_hbm.at[src_v.at[j]], bufs[i], sems[i]).wait()

    def s_start(j, i):
        pltpu.async_copy(bufs[i], acc.at[dst_v.at[j]], sems[i], add=True)

    def s_wait(j, i):
        pltpu.make_async_copy(bufs[i], acc.at[dst_v.at[j]], sems[i]).wait()

    for h in range(NSLAB):
        pltpu.sync_copy(ei_hbm.at[0, wid, h], src_v)
        pltpu.sync_copy(ei_hbm.at[1, wid, h], dst_v)
        g_start(0, 0)
        g_start(1, 1)

        def rot(m, carry):
            for i in range(NB):
                j = NB * m + i
                iw = (i + 2) % NB
                if i < NB - 2:
                    @pl.when(m > 0)
                    def _(j=j, iw=iw):
                        s_wait(j - (NB - 2), iw)
                    g_start(j + 2, iw)
                else:
                    s_wait(j - (NB - 2), iw)
                    if i == NB - 2:
                        g_start(j + 2, iw)
                    else:
                        @pl.when(m < NQ - 1)
                        def _(j=j, iw=iw):
                            g_start(j + 2, iw)
                g_wait(j, i)
                s_start(j, i)
            return carry

        lax.fori_loop(0, NQ, rot, 0)
        jt = SLAB - 1
        g_wait(jt, 0)
        s_start(jt, 0)
        s_wait(jt - 2, (jt - 2) % NB)
        s_wait(jt - 1, (jt - 1) % NB)
        s_wait(jt, 0)

    plsc.subcore_barrier()
    @pl.when(sid < 10)
    def _():
        for i in range(25):
            pltpu.sync_copy(acc.at[pl.ds(r0 + i * 40, 40)],
                            b0.at[pl.ds(0, 40)])
            pltpu.sync_copy(b0.at[pl.ds(0, 40)],
                            out_hbm.at[cid, pl.ds(r0 + i * 40, 40)])


def _sc_agg(ei5, y, zeros_nd):
    f = pl.kernel(
        _agg_body,
        out_type=jax.ShapeDtypeStruct((NC, N, D), jnp.float32),
        mesh=_sc_mesh(),
        scratch_types=[
            pltpu.VMEM_SHARED((N, D), jnp.float32),
            pltpu.VMEM((SLAB, B), jnp.int32),
            pltpu.VMEM((SLAB, B), jnp.int32),
            pltpu.VMEM((B, D), jnp.float32),
            pltpu.VMEM((B, D), jnp.float32),
            pltpu.VMEM((B, D), jnp.float32),
            pltpu.VMEM((B, D), jnp.float32),
            pltpu.SemaphoreType.DMA,
            pltpu.SemaphoreType.DMA,
            pltpu.SemaphoreType.DMA,
            pltpu.SemaphoreType.DMA,
        ],
    )
    return f(ei5, y, zeros_nd)


_RB = 1000


def _first_body(degT_ref, x_ref, W_ref, dis_ref, y_ref):
    deg = degT_ref[:, 0:1] + degT_ref[:, 1:2] + 1.0
    dis = lax.rsqrt(deg)
    dis_ref[...] = dis
    y_ref[...] = dis * jnp.dot(x_ref[...], W_ref[...],
                               preferred_element_type=jnp.float32)


def _tc_first(degT, x, W1):
    grid = N // _RB
    return pl.pallas_call(
        _first_body,
        grid=(grid,),
        in_specs=[
            pl.BlockSpec((_RB, NC), lambda i: (i, 0)),
            pl.BlockSpec((_RB, D), lambda i: (i, 0)),
            pl.BlockSpec((D, D), lambda i: (0, 0)),
        ],
        out_specs=[
            pl.BlockSpec((_RB, 1), lambda i: (i, 0)),
            pl.BlockSpec((_RB, D), lambda i: (i, 0)),
        ],
        out_shape=[
            jax.ShapeDtypeStruct((N, 1), jnp.float32),
            jax.ShapeDtypeStruct((N, D), jnp.float32),
        ],
    )(degT, x, W1)


def _mid_body(agg_ref, y_ref, dis_ref, b_ref, W_ref, out_ref):
    s = agg_ref[0] + agg_ref[1] + y_ref[...]
    h = jnp.maximum(dis_ref[...] * s + b_ref[...], 0.0)
    out_ref[...] = dis_ref[...] * jnp.dot(h, W_ref[...],
                                          preferred_element_type=jnp.float32)


def _tc_mid(aggP, y, dis, b_row, W):
    grid = N // _RB
    return pl.pallas_call(
        _mid_body,
        grid=(grid,),
        in_specs=[
            pl.BlockSpec((NC, _RB, D), lambda i: (0, i, 0)),
            pl.BlockSpec((_RB, D), lambda i: (i, 0)),
            pl.BlockSpec((_RB, 1), lambda i: (i, 0)),
            pl.BlockSpec((1, D), lambda i: (0, 0)),
            pl.BlockSpec((D, D), lambda i: (0, 0)),
        ],
        out_specs=pl.BlockSpec((_RB, D), lambda i: (i, 0)),
        out_shape=jax.ShapeDtypeStruct((N, D), jnp.float32),
    )(aggP, y, dis, b_row, W)


def _final_body(agg_ref, y_ref, dis_ref, b_ref, batch_ref, hlr_ref, std_ref,
                Wf1a_ref, Wf1b_ref, Wf1c_ref, bf1_ref, Wf2_ref, bf2_ref,
                out_ref):
    s = agg_ref[0] + agg_ref[1] + y_ref[...]
    h = jnp.maximum(dis_ref[...] * s + b_ref[...], 0.0)
    gids = lax.broadcasted_iota(jnp.int32, (G, N), 0)
    mask = (gids == batch_ref[...]).astype(jnp.float32)
    sums = jnp.dot(mask, h, preferred_element_type=jnp.float32)
    cnt = jnp.dot(mask, jnp.ones((N, 1), jnp.float32),
                  preferred_element_type=jnp.float32)
    pooled = sums / jnp.maximum(cnt, 1.0)
    z = (jnp.dot(pooled, Wf1a_ref[...], preferred_element_type=jnp.float32)
         + hlr_ref[...] * Wf1b_ref[...]
         + std_ref[...] * Wf1c_ref[...]
         + bf1_ref[...])
    z = jnp.maximum(z, 0.0)
    out_ref[...] = (jnp.dot(z, Wf2_ref[...], preferred_element_type=jnp.float32)
                    + bf2_ref[...])


def _tc_final(aggP, y, dis, b_row, batch_row, hlr, std,
              Wf1a, Wf1b, Wf1c, bf1_row, Wf2, bf2_row):
    return pl.pallas_call(
        _final_body,
        out_shape=jax.ShapeDtypeStruct((G, D), jnp.float32),
    )(aggP, y, dis, b_row, batch_row, hlr, std,
      Wf1a, Wf1b, Wf1c, bf1_row, Wf2, bf2_row)


def kernel(x, edge_index, batch, hlr, std,
           W1, b1, W2, b2, W3, b3, Wf1, bf1, Wf2, bf2):
    ei5 = edge_index.reshape(2, NW, NSLAB, SLAB, B)
    ei4 = edge_index.reshape(2, NW, DK, DB)
    zeros_40d = jnp.zeros((40, D), jnp.float32)
    zeros_dc = jnp.zeros((DEG_CHUNK,), jnp.float32)
    ones_b = jnp.ones((DB,), jnp.float32)

    degP = _sc_degree(ei4, ones_b, zeros_dc).reshape(NC, N)
    dis, y1 = _tc_first(degP.T, x, W1)

    a1 = _sc_agg(ei5, y1, zeros_40d)
    y2 = _tc_mid(a1, y1, dis, b1.reshape(1, D), W2)
    a2 = _sc_agg(ei5, y2, zeros_40d)
    y3 = _tc_mid(a2, y2, dis, b2.reshape(1, D), W3)
    a3 = _sc_agg(ei5, y3, zeros_40d)

    out = _tc_final(a3, y3, dis, b3.reshape(1, D), batch.reshape(1, N),
                    hlr, std,
                    Wf1[:D], Wf1[D:D + 1], Wf1[D + 1:D + 2],
                    bf1.reshape(1, D), Wf2, bf2.reshape(1, D))
    return out


# speedup vs baseline: 1.1365x; 1.0717x over previous
import functools

import jax
import jax.numpy as jnp
from jax import lax
from jax.experimental import pallas as pl
from jax.experimental.pallas import tpu as pltpu
from jax.experimental.pallas import tpu_sc as plsc

N = 10000
E = 320000
G = 16
D = 128

NC = 2
NS = 16
NW = NC * NS
B = 80
K = E // (NW * B)
NB = 4
SLAB = 25
NSLAB = K // SLAB
NQ = SLAB // NB
DB = 125
DK = E // (NW * DB)
ROWS_PER_TILE = N // NS
DEG_CHUNK = 1000


def _sc_mesh():
    return plsc.VectorSubcoreMesh(core_axis_name="c", subcore_axis_name="s")


def _deg_body(dst_hbm, ones_hbm, zeros_hbm, out_hbm, acc, dst_v, ones_v,
              zbuf, sem):
    cid = lax.axis_index("c")
    sid = lax.axis_index("s")
    wid = sid * NC + cid
    @pl.when(sid < 10)
    def _():
        pltpu.sync_copy(zeros_hbm, zbuf)
        pltpu.sync_copy(zbuf, acc.at[pl.ds(sid * DEG_CHUNK, DEG_CHUNK)])
    pltpu.sync_copy(dst_hbm.at[1, wid], dst_v)
    pltpu.sync_copy(ones_hbm, ones_v)
    plsc.subcore_barrier()

    def chunk(j, carry):
        pltpu.async_copy(ones_v, acc.at[dst_v.at[j]], sem, add=True).wait()
        return carry

    lax.fori_loop(0, DK, chunk, 0)
    plsc.subcore_barrier()
    @pl.when(sid < 10)
    def _():
        pltpu.sync_copy(acc.at[pl.ds(sid * DEG_CHUNK, DEG_CHUNK)], zbuf)
        pltpu.sync_copy(zbuf,
                        out_hbm.at[pl.ds(cid * N + sid * DEG_CHUNK, DEG_CHUNK)])


def _sc_degree(dst3, ones_b, zeros_n):
    f = pl.kernel(
        _deg_body,
        out_type=jax.ShapeDtypeStruct((NC * N,), jnp.float32),
        mesh=_sc_mesh(),
        scratch_types=[
            pltpu.VMEM_SHARED((N,), jnp.float32),
            pltpu.VMEM((DK, DB), jnp.int32),
            pltpu.VMEM((DB,), jnp.float32),
            pltpu.VMEM((DEG_CHUNK,), jnp.float32),
            pltpu.SemaphoreType.DMA,
        ],
    )
    return f(dst3, ones_b, zeros_n)


def _agg_body(ei_hbm, y_hbm, zeros_hbm, out_hbm,
              acc, src_v, dst_v, b0, b1, b2, b3, m0, m1, m2, m3):
    bufs = (b0, b1, b2, b3)
    sems = (m0, m1, m2, m3)
    cid = lax.axis_index("c")
    sid = lax.axis_index("s")
    wid = sid * NC + cid
    base = sid * 640
    pltpu.sync_copy(zeros_hbm, b0)

    def init_rows(nq):
        for q in range(nq):
            pltpu.async_copy(b0, acc.at[pl.ds(base + q * 80, 80)], m0)
        for q in range(nq):
            pltpu.make_async_copy(b0, acc.at[pl.ds(base + q * 80, 80)],
                                  m0).wait()

    @pl.when(sid < 15)
    def _():
        init_rows(8)
    @pl.when(sid == 15)
    def _():
        init_rows(5)
    plsc.subcore_barrier()

    def g_start(j, i):
        pltpu.async_copy(y_hbm.at[src_v.at[j]], bufs[i], sems[i])

    def g_wait(j, i):
        pltpu.make_async_copy(y_hbm.at[src_v.at[j]], bufs[i], sems[i]).wait()

    def s_start(j, i):
        pltpu.async_copy(bufs[i], acc.at[dst_v.at[j]], sems[i], add=True)

    def s_wait(j, i):
        pltpu.make_async_copy(bufs[i], acc.at[dst_v.at[j]], sems[i]).wait()

    for h in range(NSLAB):
        pltpu.sync_copy(ei_hbm.at[0, wid, h], src_v)
        pltpu.sync_copy(ei_hbm.at[1, wid, h], dst_v)
        g_start(0, 0)
        g_start(1, 1)

        def rot(m, carry):
            for i in range(NB):
                j = NB * m + i
                iw = (i + 2) % NB
                if i < NB - 2:
                    @pl.when(m > 0)
                    def _(j=j, iw=iw):
                        s_wait(j - (NB - 2), iw)
                    g_start(j + 2, iw)
                else:
                    s_wait(j - (NB - 2), iw)
                    if i == NB - 2:
                        g_start(j + 2, iw)
                    else:
                        @pl.when(m < NQ - 1)
                        def _(j=j, iw=iw):
                            g_start(j + 2, iw)
                g_wait(j, i)
                s_start(j, i)
            return carry

        lax.fori_loop(0, NQ, rot, 0)
        jt = SLAB - 1
        g_wait(jt, 0)
        s_start(jt, 0)
        s_wait(jt - 2, (jt - 2) % NB)
        s_wait(jt - 1, (jt - 1) % NB)
        s_wait(jt, 0)

    plsc.subcore_barrier()

    def copy_out(nq):
        def wait_hbm(q):
            buf, sem = (b0, m0) if q % 2 == 0 else (b1, m1)
            pltpu.make_async_copy(
                buf, out_hbm.at[cid, pl.ds(base + q * 80, 80)], sem).wait()
        for q in range(nq):
            buf, sem = (b0, m0) if q % 2 == 0 else (b1, m1)
            if q >= 2:
                wait_hbm(q - 2)
            pltpu.sync_copy(acc.at[pl.ds(base + q * 80, 80)], buf)
            pltpu.async_copy(buf, out_hbm.at[cid, pl.ds(base + q * 80, 80)],
                             sem)
        for q in range(max(nq - 2, 0), nq):
            wait_hbm(q)

    @pl.when(sid < 15)
    def _():
        copy_out(8)
    @pl.when(sid == 15)
    def _():
        copy_out(5)


def _sc_agg(ei5, y, zeros_nd):
    f = pl.kernel(
        _agg_body,
        out_type=jax.ShapeDtypeStruct((NC, N, D), jnp.float32),
        mesh=_sc_mesh(),
        scratch_types=[
            pltpu.VMEM_SHARED((N, D), jnp.float32),
            pltpu.VMEM((SLAB, B), jnp.int32),
            pltpu.VMEM((SLAB, B), jnp.int32),
            pltpu.VMEM((B, D), jnp.float32),
            pltpu.VMEM((B, D), jnp.float32),
            pltpu.VMEM((B, D), jnp.float32),
            pltpu.VMEM((B, D), jnp.float32),
            pltpu.SemaphoreType.DMA,
            pltpu.SemaphoreType.DMA,
            pltpu.SemaphoreType.DMA,
            pltpu.SemaphoreType.DMA,
        ],
    )
    return f(ei5, y, zeros_nd)


_RB = 1000


def _first_body(degT_ref, x_ref, W_ref, dis_ref, y_ref):
    deg = degT_ref[:, 0:1] + degT_ref[:, 1:2] + 1.0
    dis = lax.rsqrt(deg)
    dis_ref[...] = dis
    y_ref[...] = dis * jnp.dot(x_ref[...], W_ref[...],
                               preferred_element_type=jnp.float32)


def _tc_first(degT, x, W1):
    grid = N // _RB
    return pl.pallas_call(
        _first_body,
        grid=(grid,),
        in_specs=[
            pl.BlockSpec((_RB, NC), lambda i: (i, 0)),
            pl.BlockSpec((_RB, D), lambda i: (i, 0)),
            pl.BlockSpec((D, D), lambda i: (0, 0)),
        ],
        out_specs=[
            pl.BlockSpec((_RB, 1), lambda i: (i, 0)),
            pl.BlockSpec((_RB, D), lambda i: (i, 0)),
        ],
        out_shape=[
            jax.ShapeDtypeStruct((N, 1), jnp.float32),
            jax.ShapeDtypeStruct((N, D), jnp.float32),
        ],
    )(degT, x, W1)


def _mid_body(agg_ref, y_ref, dis_ref, b_ref, W_ref, out_ref):
    s = agg_ref[0] + agg_ref[1] + y_ref[...]
    h = jnp.maximum(dis_ref[...] * s + b_ref[...], 0.0)
    out_ref[...] = dis_ref[...] * jnp.dot(h, W_ref[...],
                                          preferred_element_type=jnp.float32)


def _tc_mid(aggP, y, dis, b_row, W):
    grid = N // _RB
    return pl.pallas_call(
        _mid_body,
        grid=(grid,),
        in_specs=[
            pl.BlockSpec((NC, _RB, D), lambda i: (0, i, 0)),
            pl.BlockSpec((_RB, D), lambda i: (i, 0)),
            pl.BlockSpec((_RB, 1), lambda i: (i, 0)),
            pl.BlockSpec((1, D), lambda i: (0, 0)),
            pl.BlockSpec((D, D), lambda i: (0, 0)),
        ],
        out_specs=pl.BlockSpec((_RB, D), lambda i: (i, 0)),
        out_shape=jax.ShapeDtypeStruct((N, D), jnp.float32),
    )(aggP, y, dis, b_row, W)


def _final_body(agg_ref, y_ref, dis_ref, b_ref, batch_ref, hlr_ref, std_ref,
                Wf1a_ref, Wf1b_ref, Wf1c_ref, bf1_ref, Wf2_ref, bf2_ref,
                out_ref):
    s = agg_ref[0] + agg_ref[1] + y_ref[...]
    h = jnp.maximum(dis_ref[...] * s + b_ref[...], 0.0)
    gids = lax.broadcasted_iota(jnp.int32, (G, N), 0)
    mask = (gids == batch_ref[...]).astype(jnp.float32)
    sums = jnp.dot(mask, h, preferred_element_type=jnp.float32)
    cnt = jnp.dot(mask, jnp.ones((N, 1), jnp.float32),
                  preferred_element_type=jnp.float32)
    pooled = sums / jnp.maximum(cnt, 1.0)
    z = (jnp.dot(pooled, Wf1a_ref[...], preferred_element_type=jnp.float32)
         + hlr_ref[...] * Wf1b_ref[...]
         + std_ref[...] * Wf1c_ref[...]
         + bf1_ref[...])
    z = jnp.maximum(z, 0.0)
    out_ref[...] = (jnp.dot(z, Wf2_ref[...], preferred_element_type=jnp.float32)
                    + bf2_ref[...])


def _tc_final(aggP, y, dis, b_row, batch_row, hlr, std,
              Wf1a, Wf1b, Wf1c, bf1_row, Wf2, bf2_row):
    return pl.pallas_call(
        _final_body,
        out_shape=jax.ShapeDtypeStruct((G, D), jnp.float32),
    )(aggP, y, dis, b_row, batch_row, hlr, std,
      Wf1a, Wf1b, Wf1c, bf1_row, Wf2, bf2_row)


def kernel(x, edge_index, batch, hlr, std,
           W1, b1, W2, b2, W3, b3, Wf1, bf1, Wf2, bf2):
    ei5 = edge_index.reshape(2, NW, NSLAB, SLAB, B)
    ei4 = edge_index.reshape(2, NW, DK, DB)
    zeros_80d = jnp.zeros((80, D), jnp.float32)
    zeros_dc = jnp.zeros((DEG_CHUNK,), jnp.float32)
    ones_b = jnp.ones((DB,), jnp.float32)

    degP = _sc_degree(ei4, ones_b, zeros_dc).reshape(NC, N)
    dis, y1 = _tc_first(degP.T, x, W1)

    a1 = _sc_agg(ei5, y1, zeros_80d)
    y2 = _tc_mid(a1, y1, dis, b1.reshape(1, D), W2)
    a2 = _sc_agg(ei5, y2, zeros_80d)
    y3 = _tc_mid(a2, y2, dis, b2.reshape(1, D), W3)
    a3 = _sc_agg(ei5, y3, zeros_80d)

    out = _tc_final(a3, y3, dis, b3.reshape(1, D), batch.reshape(1, N),
                    hlr, std,
                    Wf1[:D], Wf1[D:D + 1], Wf1[D + 1:D + 2],
                    bf1.reshape(1, D), Wf2, bf2.reshape(1, D))
    return out


# speedup vs baseline: 1.1581x; 1.0190x over previous
import functools

import jax
import jax.numpy as jnp
from jax import lax
from jax.experimental import pallas as pl
from jax.experimental.pallas import tpu as pltpu
from jax.experimental.pallas import tpu_sc as plsc

N = 10000
E = 320000
G = 16
D = 128

NC = 2
NS = 16
NW = NC * NS
B = 80
K = E // (NW * B)
NB = 4
SLAB = 25
NSLAB = K // SLAB
NQ = SLAB // NB
DB = 125
DK = E // (NW * DB)
ROWS_PER_TILE = N // NS
DEG_CHUNK = 1000


def _sc_mesh():
    return plsc.VectorSubcoreMesh(core_axis_name="c", subcore_axis_name="s")


def _deg_body(dst_hbm, ones_hbm, zeros_hbm, out_hbm, acc, dst_v, ones_v,
              zbuf, sem):
    cid = lax.axis_index("c")
    sid = lax.axis_index("s")
    wid = sid * NC + cid
    @pl.when(sid < 10)
    def _():
        pltpu.sync_copy(zeros_hbm, zbuf)
        pltpu.sync_copy(zbuf, acc.at[pl.ds(sid * DEG_CHUNK, DEG_CHUNK)])
    pltpu.sync_copy(dst_hbm.at[1, wid], dst_v)
    pltpu.sync_copy(ones_hbm, ones_v)
    plsc.subcore_barrier()

    def chunk(j, carry):
        pltpu.async_copy(ones_v, acc.at[dst_v.at[j]], sem, add=True)
        return carry

    lax.fori_loop(0, DK, chunk, 0)

    def drain(j, carry):
        pltpu.make_async_copy(ones_v, acc.at[dst_v.at[j]], sem).wait()
        return carry

    lax.fori_loop(0, DK, drain, 0)
    plsc.subcore_barrier()
    @pl.when(sid < 10)
    def _():
        pltpu.sync_copy(acc.at[pl.ds(sid * DEG_CHUNK, DEG_CHUNK)], zbuf)
        pltpu.sync_copy(zbuf,
                        out_hbm.at[pl.ds(cid * N + sid * DEG_CHUNK, DEG_CHUNK)])


def _sc_degree(dst3, ones_b, zeros_n):
    f = pl.kernel(
        _deg_body,
        out_type=jax.ShapeDtypeStruct((NC * N,), jnp.float32),
        mesh=_sc_mesh(),
        scratch_types=[
            pltpu.VMEM_SHARED((N,), jnp.float32),
            pltpu.VMEM((DK, DB), jnp.int32),
            pltpu.VMEM((DB,), jnp.float32),
            pltpu.VMEM((DEG_CHUNK,), jnp.float32),
            pltpu.SemaphoreType.DMA,
        ],
    )
    return f(dst3, ones_b, zeros_n)


def _agg_body(ei_hbm, y_hbm, zeros_hbm, out_hbm,
              acc, src_v, dst_v, b0, b1, b2, b3, m0, m1, m2, m3):
    bufs = (b0, b1, b2, b3)
    sems = (m0, m1, m2, m3)
    cid = lax.axis_index("c")
    sid = lax.axis_index("s")
    wid = sid * NC + cid
    base = sid * 640

    def g_start(j, i):
        pltpu.async_copy(y_hbm.at[src_v.at[j]], bufs[i], sems[i])

    def g_wait(j, i):
        pltpu.make_async_copy(y_hbm.at[src_v.at[j]], bufs[i], sems[i]).wait()

    def s_start(j, i):
        pltpu.async_copy(bufs[i], acc.at[dst_v.at[j]], sems[i], add=True)

    def s_wait(j, i):
        pltpu.make_async_copy(bufs[i], acc.at[dst_v.at[j]], sems[i]).wait()

    pltpu.sync_copy(ei_hbm.at[0, wid, 0], src_v)
    pltpu.sync_copy(ei_hbm.at[1, wid, 0], dst_v)
    g_start(0, 0)
    g_start(1, 1)
    pltpu.sync_copy(zeros_hbm, b3)

    def init_rows(nq):
        for q in range(nq):
            pltpu.async_copy(b3, acc.at[pl.ds(base + q * 80, 80)], m3)
        for q in range(nq):
            pltpu.make_async_copy(b3, acc.at[pl.ds(base + q * 80, 80)],
                                  m3).wait()

    @pl.when(sid < 15)
    def _():
        init_rows(8)
    @pl.when(sid == 15)
    def _():
        init_rows(5)
    plsc.subcore_barrier()

    for h in range(NSLAB):
        if h > 0:
            pltpu.sync_copy(ei_hbm.at[0, wid, h], src_v)
            pltpu.sync_copy(ei_hbm.at[1, wid, h], dst_v)
            g_start(0, 0)
            g_start(1, 1)

        def rot(m, carry):
            for i in range(NB):
                j = NB * m + i
                iw = (i + 2) % NB
                if i < NB - 2:
                    @pl.when(m > 0)
                    def _(j=j, iw=iw):
                        s_wait(j - (NB - 2), iw)
                    g_start(j + 2, iw)
                else:
                    s_wait(j - (NB - 2), iw)
                    if i == NB - 2:
                        g_start(j + 2, iw)
                    else:
                        @pl.when(m < NQ - 1)
                        def _(j=j, iw=iw):
                            g_start(j + 2, iw)
                g_wait(j, i)
                s_start(j, i)
            return carry

        lax.fori_loop(0, NQ, rot, 0)
        jt = SLAB - 1
        g_wait(jt, 0)
        s_start(jt, 0)
        s_wait(jt - 2, (jt - 2) % NB)
        s_wait(jt - 1, (jt - 1) % NB)
        s_wait(jt, 0)

    plsc.subcore_barrier()

    def copy_out(nq):
        def wait_hbm(q):
            buf, sem = (b0, m0) if q % 2 == 0 else (b1, m1)
            pltpu.make_async_copy(
                buf, out_hbm.at[cid, pl.ds(base + q * 80, 80)], sem).wait()
        for q in range(nq):
            buf, sem = (b0, m0) if q % 2 == 0 else (b1, m1)
            if q >= 2:
                wait_hbm(q - 2)
            pltpu.sync_copy(acc.at[pl.ds(base + q * 80, 80)], buf)
            pltpu.async_copy(buf, out_hbm.at[cid, pl.ds(base + q * 80, 80)],
                             sem)
        for q in range(max(nq - 2, 0), nq):
            wait_hbm(q)

    @pl.when(sid < 15)
    def _():
        copy_out(8)
    @pl.when(sid == 15)
    def _():
        copy_out(5)


def _sc_agg(ei5, y, zeros_nd):
    f = pl.kernel(
        _agg_body,
        out_type=jax.ShapeDtypeStruct((NC, N, D), jnp.float32),
        mesh=_sc_mesh(),
        scratch_types=[
            pltpu.VMEM_SHARED((N, D), jnp.float32),
            pltpu.VMEM((SLAB, B), jnp.int32),
            pltpu.VMEM((SLAB, B), jnp.int32),
            pltpu.VMEM((B, D), jnp.float32),
            pltpu.VMEM((B, D), jnp.float32),
            pltpu.VMEM((B, D), jnp.float32),
            pltpu.VMEM((B, D), jnp.float32),
            pltpu.SemaphoreType.DMA,
            pltpu.SemaphoreType.DMA,
            pltpu.SemaphoreType.DMA,
            pltpu.SemaphoreType.DMA,
        ],
    )
    return f(ei5, y, zeros_nd)


_RB = 1000


def _first_body(degT_ref, x_ref, W_ref, dis_ref, y_ref):
    deg = degT_ref[:, 0:1] + degT_ref[:, 1:2] + 1.0
    dis = lax.rsqrt(deg)
    dis_ref[...] = dis
    y_ref[...] = dis * jnp.dot(x_ref[...], W_ref[...],
                               preferred_element_type=jnp.float32)


def _tc_first(degT, x, W1):
    grid = N // _RB
    return pl.pallas_call(
        _first_body,
        grid=(grid,),
        in_specs=[
            pl.BlockSpec((_RB, NC), lambda i: (i, 0)),
            pl.BlockSpec((_RB, D), lambda i: (i, 0)),
            pl.BlockSpec((D, D), lambda i: (0, 0)),
        ],
        out_specs=[
            pl.BlockSpec((_RB, 1), lambda i: (i, 0)),
            pl.BlockSpec((_RB, D), lambda i: (i, 0)),
        ],
        out_shape=[
            jax.ShapeDtypeStruct((N, 1), jnp.float32),
            jax.ShapeDtypeStruct((N, D), jnp.float32),
        ],
    )(degT, x, W1)


def _mid_body(agg_ref, y_ref, dis_ref, b_ref, W_ref, out_ref):
    s = agg_ref[0] + agg_ref[1] + y_ref[...]
    h = jnp.maximum(dis_ref[...] * s + b_ref[...], 0.0)
    out_ref[...] = dis_ref[...] * jnp.dot(h, W_ref[...],
                                          preferred_element_type=jnp.float32)


def _tc_mid(aggP, y, dis, b_row, W):
    grid = N // _RB
    return pl.pallas_call(
        _mid_body,
        grid=(grid,),
        in_specs=[
            pl.BlockSpec((NC, _RB, D), lambda i: (0, i, 0)),
            pl.BlockSpec((_RB, D), lambda i: (i, 0)),
            pl.BlockSpec((_RB, 1), lambda i: (i, 0)),
            pl.BlockSpec((1, D), lambda i: (0, 0)),
            pl.BlockSpec((D, D), lambda i: (0, 0)),
        ],
        out_specs=pl.BlockSpec((_RB, D), lambda i: (i, 0)),
        out_shape=jax.ShapeDtypeStruct((N, D), jnp.float32),
    )(aggP, y, dis, b_row, W)


def _final_body(agg_ref, y_ref, dis_ref, b_ref, batch_ref, hlr_ref, std_ref,
                Wf1a_ref, Wf1b_ref, Wf1c_ref, bf1_ref, Wf2_ref, bf2_ref,
                out_ref):
    s = agg_ref[0] + agg_ref[1] + y_ref[...]
    h = jnp.maximum(dis_ref[...] * s + b_ref[...], 0.0)
    gids = lax.broadcasted_iota(jnp.int32, (G, N), 0)
    mask = (gids == batch_ref[...]).astype(jnp.float32)
    sums = jnp.dot(mask, h, preferred_element_type=jnp.float32)
    cnt = jnp.dot(mask, jnp.ones((N, 1), jnp.float32),
                  preferred_element_type=jnp.float32)
    pooled = sums / jnp.maximum(cnt, 1.0)
    z = (jnp.dot(pooled, Wf1a_ref[...], preferred_element_type=jnp.float32)
         + hlr_ref[...] * Wf1b_ref[...]
         + std_ref[...] * Wf1c_ref[...]
         + bf1_ref[...])
    z = jnp.maximum(z, 0.0)
    out_ref[...] = (jnp.dot(z, Wf2_ref[...], preferred_element_type=jnp.float32)
                    + bf2_ref[...])


def _tc_final(aggP, y, dis, b_row, batch_row, hlr, std,
              Wf1a, Wf1b, Wf1c, bf1_row, Wf2, bf2_row):
    return pl.pallas_call(
        _final_body,
        out_shape=jax.ShapeDtypeStruct((G, D), jnp.float32),
    )(aggP, y, dis, b_row, batch_row, hlr, std,
      Wf1a, Wf1b, Wf1c, bf1_row, Wf2, bf2_row)


def kernel(x, edge_index, batch, hlr, std,
           W1, b1, W2, b2, W3, b3, Wf1, bf1, Wf2, bf2):
    ei5 = edge_index.reshape(2, NW, NSLAB, SLAB, B)
    ei4 = edge_index.reshape(2, NW, DK, DB)
    zeros_80d = jnp.zeros((80, D), jnp.float32)
    zeros_dc = jnp.zeros((DEG_CHUNK,), jnp.float32)
    ones_b = jnp.ones((DB,), jnp.float32)

    degP = _sc_degree(ei4, ones_b, zeros_dc).reshape(NC, N)
    dis, y1 = _tc_first(degP.T, x, W1)

    a1 = _sc_agg(ei5, y1, zeros_80d)
    y2 = _tc_mid(a1, y1, dis, b1.reshape(1, D), W2)
    a2 = _sc_agg(ei5, y2, zeros_80d)
    y3 = _tc_mid(a2, y2, dis, b2.reshape(1, D), W3)
    a3 = _sc_agg(ei5, y3, zeros_80d)

    out = _tc_final(a3, y3, dis, b3.reshape(1, D), batch.reshape(1, N),
                    hlr, std,
                    Wf1[:D], Wf1[D:D + 1], Wf1[D + 1:D + 2],
                    bf1.reshape(1, D), Wf2, bf2.reshape(1, D))
    return out


# speedup vs baseline: 1.1659x; 1.0068x over previous
import functools

import jax
import jax.numpy as jnp
from jax import lax
from jax.experimental import pallas as pl
from jax.experimental.pallas import tpu as pltpu
from jax.experimental.pallas import tpu_sc as plsc

N = 10000
E = 320000
G = 16
D = 128

NC = 2
NS = 16
NW = NC * NS
B = 80
K = E // (NW * B)
NB = 4
SLAB = 25
NSLAB = K // SLAB
NQ = SLAB // NB
DB = 125
DK = E // (NW * DB)
ROWS_PER_TILE = N // NS
DEG_CHUNK = 1000


def _sc_mesh():
    return plsc.VectorSubcoreMesh(core_axis_name="c", subcore_axis_name="s")


def _deg_body(ei_hbm, ones_hbm, zeros_hbm, out_hbm, acc, dst_v, ones_v,
              zbuf, sem):
    cid = lax.axis_index("c")
    sid = lax.axis_index("s")
    wid = sid * NC + cid
    @pl.when(sid < 10)
    def _():
        pltpu.sync_copy(zeros_hbm, zbuf)
        pltpu.sync_copy(zbuf, acc.at[pl.ds(sid * DEG_CHUNK, DEG_CHUNK)])
    pltpu.sync_copy(ei_hbm.at[1, wid], dst_v)
    pltpu.sync_copy(ones_hbm, ones_v)
    plsc.subcore_barrier()

    def chunk(j, carry):
        h, jj = j // SLAB, j % SLAB
        pltpu.async_copy(ones_v, acc.at[dst_v.at[h, jj]], sem, add=True)
        return carry

    lax.fori_loop(0, K, chunk, 0)

    def drain(j, carry):
        h, jj = j // SLAB, j % SLAB
        pltpu.make_async_copy(ones_v, acc.at[dst_v.at[h, jj]], sem).wait()
        return carry

    lax.fori_loop(0, K, drain, 0)
    plsc.subcore_barrier()
    @pl.when(sid < 10)
    def _():
        pltpu.sync_copy(acc.at[pl.ds(sid * DEG_CHUNK, DEG_CHUNK)], zbuf)
        pltpu.sync_copy(zbuf,
                        out_hbm.at[pl.ds(cid * N + sid * DEG_CHUNK, DEG_CHUNK)])


def _sc_degree(ei5, ones_b, zeros_n):
    f = pl.kernel(
        _deg_body,
        out_type=jax.ShapeDtypeStruct((NC * N,), jnp.float32),
        mesh=_sc_mesh(),
        scratch_types=[
            pltpu.VMEM_SHARED((N,), jnp.float32),
            pltpu.VMEM((NSLAB, SLAB, B), jnp.int32),
            pltpu.VMEM((B,), jnp.float32),
            pltpu.VMEM((DEG_CHUNK,), jnp.float32),
            pltpu.SemaphoreType.DMA,
        ],
    )
    return f(ei5, ones_b, zeros_n)


def _agg_body(ei_hbm, y_hbm, zeros_hbm, out_hbm,
              acc, src_v, dst_v, b0, b1, b2, b3, m0, m1, m2, m3):
    bufs = (b0, b1, b2, b3)
    sems = (m0, m1, m2, m3)
    cid = lax.axis_index("c")
    sid = lax.axis_index("s")
    wid = sid * NC + cid
    base = sid * 640

    def g_start(j, i):
        pltpu.async_copy(y_hbm.at[src_v.at[j]], bufs[i], sems[i])

    def g_wait(j, i):
        pltpu.make_async_copy(y_hbm.at[src_v.at[j]], bufs[i], sems[i]).wait()

    def s_start(j, i):
        pltpu.async_copy(bufs[i], acc.at[dst_v.at[j]], sems[i], add=True)

    def s_wait(j, i):
        pltpu.make_async_copy(bufs[i], acc.at[dst_v.at[j]], sems[i]).wait()

    pltpu.sync_copy(ei_hbm.at[0, wid, 0], src_v)
    pltpu.sync_copy(ei_hbm.at[1, wid, 0], dst_v)
    g_start(0, 0)
    g_start(1, 1)
    pltpu.sync_copy(zeros_hbm, b3)

    def init_rows(nq):
        for q in range(nq):
            pltpu.async_copy(b3, acc.at[pl.ds(base + q * 80, 80)], m3)
        for q in range(nq):
            pltpu.make_async_copy(b3, acc.at[pl.ds(base + q * 80, 80)],
                                  m3).wait()

    @pl.when(sid < 15)
    def _():
        init_rows(8)
    @pl.when(sid == 15)
    def _():
        init_rows(5)
    plsc.subcore_barrier()

    for h in range(NSLAB):
        if h > 0:
            pltpu.sync_copy(ei_hbm.at[0, wid, h], src_v)
            pltpu.sync_copy(ei_hbm.at[1, wid, h], dst_v)
            g_start(0, 0)
            g_start(1, 1)

        def rot(m, carry):
            for i in range(NB):
                j = NB * m + i
                iw = (i + 2) % NB
                if i < NB - 2:
                    @pl.when(m > 0)
                    def _(j=j, iw=iw):
                        s_wait(j - (NB - 2), iw)
                    g_start(j + 2, iw)
                else:
                    s_wait(j - (NB - 2), iw)
                    if i == NB - 2:
                        g_start(j + 2, iw)
                    else:
                        @pl.when(m < NQ - 1)
                        def _(j=j, iw=iw):
                            g_start(j + 2, iw)
                g_wait(j, i)
                s_start(j, i)
            return carry

        lax.fori_loop(0, NQ, rot, 0)
        jt = SLAB - 1
        g_wait(jt, 0)
        s_start(jt, 0)
        s_wait(jt - 2, (jt - 2) % NB)
        s_wait(jt - 1, (jt - 1) % NB)
        s_wait(jt, 0)

    plsc.subcore_barrier()

    def copy_out(nq):
        def wait_hbm(q):
            buf, sem = (b0, m0) if q % 2 == 0 else (b1, m1)
            pltpu.make_async_copy(
                buf, out_hbm.at[cid, pl.ds(base + q * 80, 80)], sem).wait()
        for q in range(nq):
            buf, sem = (b0, m0) if q % 2 == 0 else (b1, m1)
            if q >= 2:
                wait_hbm(q - 2)
            pltpu.sync_copy(acc.at[pl.ds(base + q * 80, 80)], buf)
            pltpu.async_copy(buf, out_hbm.at[cid, pl.ds(base + q * 80, 80)],
                             sem)
        for q in range(max(nq - 2, 0), nq):
            wait_hbm(q)

    @pl.when(sid < 15)
    def _():
        copy_out(8)
    @pl.when(sid == 15)
    def _():
        copy_out(5)


def _sc_agg(ei5, y, zeros_nd):
    f = pl.kernel(
        _agg_body,
        out_type=jax.ShapeDtypeStruct((NC, N, D), jnp.float32),
        mesh=_sc_mesh(),
        scratch_types=[
            pltpu.VMEM_SHARED((N, D), jnp.float32),
            pltpu.VMEM((SLAB, B), jnp.int32),
            pltpu.VMEM((SLAB, B), jnp.int32),
            pltpu.VMEM((B, D), jnp.float32),
            pltpu.VMEM((B, D), jnp.float32),
            pltpu.VMEM((B, D), jnp.float32),
            pltpu.VMEM((B, D), jnp.float32),
            pltpu.SemaphoreType.DMA,
            pltpu.SemaphoreType.DMA,
            pltpu.SemaphoreType.DMA,
            pltpu.SemaphoreType.DMA,
        ],
    )
    return f(ei5, y, zeros_nd)


_RB = 1000


def _first_body(degT_ref, x_ref, W_ref, dis_ref, y_ref):
    deg = degT_ref[:, 0:1] + degT_ref[:, 1:2] + 1.0
    dis = lax.rsqrt(deg)
    dis_ref[...] = dis
    y_ref[...] = dis * jnp.dot(x_ref[...], W_ref[...],
                               preferred_element_type=jnp.float32)


def _tc_first(degT, x, W1):
    grid = N // _RB
    return pl.pallas_call(
        _first_body,
        grid=(grid,),
        in_specs=[
            pl.BlockSpec((_RB, NC), lambda i: (i, 0)),
            pl.BlockSpec((_RB, D), lambda i: (i, 0)),
            pl.BlockSpec((D, D), lambda i: (0, 0)),
        ],
        out_specs=[
            pl.BlockSpec((_RB, 1), lambda i: (i, 0)),
            pl.BlockSpec((_RB, D), lambda i: (i, 0)),
        ],
        out_shape=[
            jax.ShapeDtypeStruct((N, 1), jnp.float32),
            jax.ShapeDtypeStruct((N, D), jnp.float32),
        ],
    )(degT, x, W1)


def _mid_body(agg_ref, y_ref, dis_ref, b_ref, W_ref, out_ref):
    s = agg_ref[0] + agg_ref[1] + y_ref[...]
    h = jnp.maximum(dis_ref[...] * s + b_ref[...], 0.0)
    out_ref[...] = dis_ref[...] * jnp.dot(h, W_ref[...],
                                          preferred_element_type=jnp.float32)


def _tc_mid(aggP, y, dis, b_row, W):
    grid = N // _RB
    return pl.pallas_call(
        _mid_body,
        grid=(grid,),
        in_specs=[
            pl.BlockSpec((NC, _RB, D), lambda i: (0, i, 0)),
            pl.BlockSpec((_RB, D), lambda i: (i, 0)),
            pl.BlockSpec((_RB, 1), lambda i: (i, 0)),
            pl.BlockSpec((1, D), lambda i: (0, 0)),
            pl.BlockSpec((D, D), lambda i: (0, 0)),
        ],
        out_specs=pl.BlockSpec((_RB, D), lambda i: (i, 0)),
        out_shape=jax.ShapeDtypeStruct((N, D), jnp.float32),
    )(aggP, y, dis, b_row, W)


def _final_body(agg_ref, y_ref, dis_ref, b_ref, batch_ref, hlr_ref, std_ref,
                Wf1a_ref, Wf1b_ref, Wf1c_ref, bf1_ref, Wf2_ref, bf2_ref,
                out_ref):
    s = agg_ref[0] + agg_ref[1] + y_ref[...]
    h = jnp.maximum(dis_ref[...] * s + b_ref[...], 0.0)
    gids = lax.broadcasted_iota(jnp.int32, (G, N), 0)
    mask = (gids == batch_ref[...]).astype(jnp.float32)
    sums = jnp.dot(mask, h, preferred_element_type=jnp.float32)
    cnt = jnp.dot(mask, jnp.ones((N, 1), jnp.float32),
                  preferred_element_type=jnp.float32)
    pooled = sums / jnp.maximum(cnt, 1.0)
    z = (jnp.dot(pooled, Wf1a_ref[...], preferred_element_type=jnp.float32)
         + hlr_ref[...] * Wf1b_ref[...]
         + std_ref[...] * Wf1c_ref[...]
         + bf1_ref[...])
    z = jnp.maximum(z, 0.0)
    out_ref[...] = (jnp.dot(z, Wf2_ref[...], preferred_element_type=jnp.float32)
                    + bf2_ref[...])


def _tc_final(aggP, y, dis, b_row, batch_row, hlr, std,
              Wf1a, Wf1b, Wf1c, bf1_row, Wf2, bf2_row):
    return pl.pallas_call(
        _final_body,
        out_shape=jax.ShapeDtypeStruct((G, D), jnp.float32),
    )(aggP, y, dis, b_row, batch_row, hlr, std,
      Wf1a, Wf1b, Wf1c, bf1_row, Wf2, bf2_row)


def kernel(x, edge_index, batch, hlr, std,
           W1, b1, W2, b2, W3, b3, Wf1, bf1, Wf2, bf2):
    ei5 = edge_index.reshape(2, NW, NSLAB, SLAB, B)
    zeros_80d = jnp.zeros((80, D), jnp.float32)
    zeros_dc = jnp.zeros((DEG_CHUNK,), jnp.float32)
    ones_b = jnp.ones((B,), jnp.float32)

    degP = _sc_degree(ei5, ones_b, zeros_dc).reshape(NC, N)
    dis, y1 = _tc_first(degP.T, x, W1)

    a1 = _sc_agg(ei5, y1, zeros_80d)
    y2 = _tc_mid(a1, y1, dis, b1.reshape(1, D), W2)
    a2 = _sc_agg(ei5, y2, zeros_80d)
    y3 = _tc_mid(a2, y2, dis, b2.reshape(1, D), W3)
    a3 = _sc_agg(ei5, y3, zeros_80d)

    out = _tc_final(a3, y3, dis, b3.reshape(1, D), batch.reshape(1, N),
                    hlr, std,
                    Wf1[:D], Wf1[D:D + 1], Wf1[D + 1:D + 2],
                    bf1.reshape(1, D), Wf2, bf2.reshape(1, D))
    return out
